# baseline (speedup 1.0000x reference)
import functools

import jax
import jax.numpy as jnp
from jax import lax
from jax.experimental import pallas as pl
from jax.experimental.pallas import tpu as pltpu
from jax.experimental.pallas import tpu_sc as plsc

_NC, _NS = 2, 16
_NW = _NC * _NS

_H, _W, _D = 1024, 1024, 48
_R = _H * _W
_B = 524288
_BANDS = _D // 8
_WT = _W // 128

_mesh = plsc.VectorSubcoreMesh(core_axis_name="c", subcore_axis_name="s")
_params = pltpu.CompilerParams(use_tc_tiling_on_sc=False,
                               needs_layout_passes=False)


def _wid():
    return lax.axis_index("s") * _NC + lax.axis_index("c")


@functools.partial(
    pl.kernel,
    out_type=jax.ShapeDtypeStruct((_R, _D), jnp.float32),
    mesh=_mesh,
    scratch_types=[
        pltpu.VMEM((2, _BANDS, 1024), jnp.float32),
        pltpu.VMEM((2, 128, _D), jnp.float32),
        pltpu.SemaphoreType.DMA,
        pltpu.SemaphoreType.DMA,
        pltpu.SemaphoreType.DMA,
        pltpu.SemaphoreType.DMA,
    ],
    compiler_params=_params,
)
def _transpose_kernel(tab_v, rm_out, src_v, dst_v, si0, si1, so0, so1):
    wid = _wid()
    h0 = wid * (_H // _NW)
    n_blocks = (_H // _NW) * _WT
    sin = (si0, si1)
    sout = (so0, so1)
    i16 = lax.iota(jnp.int32, 16)
    idx_a = []
    idx_b = []
    for p in range(3):
        dp = p * 16 + i16
        idx_a.append(dp // 8)
        idx_b.append((dp % 8) * 128)

    def src_slice(blk):
        h = h0 + blk // _WT
        wt = blk % _WT
        return tab_v.at[h, :, pl.ds(wt * 1024, 1024)]

    for u in (0, 1):
        pltpu.async_copy(src_slice(u), src_v.at[u], sin[u])

    def body(g, carry):
        for u in (0, 1):
            i = 2 * g + u
            pltpu.make_async_copy(src_slice(i), src_v.at[u], sin[u]).wait()

            @pl.when(g > 0)
            def _():
                pltpu.make_async_copy(
                    dst_v.at[u],
                    rm_out.at[pl.ds((h0 * _W // 128 + (i - 2)) * 128, 128)],
                    sout[u]).wait()

            def shuf(jc, c2):
                for jj in range(8):
                    j = jc * 8 + jj
                    for p in range(3):
                        vals = plsc.load_gather(
                            src_v.at[u], [idx_a[p], idx_b[p] + j])
                        dst_v[u, j, pl.ds(p * 16, 16)] = vals
                return c2
            lax.fori_loop(0, 16, shuf, 0)

            pltpu.async_copy(
                dst_v.at[u],
                rm_out.at[pl.ds((h0 * _W // 128 + i) * 128, 128)],
                sout[u])

            @pl.when(g <= (n_blocks // 2 - 2))
            def _():
                pltpu.async_copy(src_slice(i + 2), src_v.at[u], sin[u])
        return carry

    lax.fori_loop(0, n_blocks // 2, body, 0)
    for u in (0, 1):
        i = n_blocks - 2 + u
        pltpu.make_async_copy(
            dst_v.at[u],
            rm_out.at[pl.ds((h0 * _W // 128 + i) * 128, 128)],
            sout[u]).wait()


@functools.partial(
    pl.kernel,
    out_type=jax.ShapeDtypeStruct((_BANDS, _B // 128, 8, 128), jnp.float32),
    mesh=_mesh,
    scratch_types=[
        pltpu.VMEM((_B // _NW,), jnp.int32),
        pltpu.VMEM((2, 128, _D), jnp.float32),
        pltpu.VMEM((2, _BANDS, 8, 128), jnp.float32),
        pltpu.SemaphoreType.DMA,
        pltpu.SemaphoreType.DMA,
        pltpu.SemaphoreType.DMA,
        pltpu.SemaphoreType.DMA,
    ],
    compiler_params=_params,
)
def _gather_kernel(rm_tab, idx_hbm, out_v, idx_v, rows_v, dst_v,
                   sg0, sg1, so0, so1):
    wid = _wid()
    b_per_w = _B // _NW
    n_blocks = b_per_w // 128
    ot0 = wid * n_blocks
    sg = (sg0, sg1)
    so = (so0, so1)
    pltpu.sync_copy(idx_hbm.at[pl.ds(wid * b_per_w, b_per_w)], idx_v)
    i16 = lax.iota(jnp.int32, 16)

    def gather(i, u):
        return pltpu.async_copy(
            rm_tab.at[idx_v.at[pl.ds(i * 128, 128)]], rows_v.at[u], sg[u])

    for u in (0, 1):
        gather(u, u)

    def body(g, carry):
        for u in (0, 1):
            i = 2 * g + u
            pltpu.make_async_copy(
                rm_tab.at[idx_v.at[pl.ds(i * 128, 128)]],
                rows_v.at[u], sg[u]).wait()

            @pl.when(g > 0)
            def _():
                pltpu.make_async_copy(
                    dst_v.at[u], out_v.at[:, ot0 + i - 2], so[u]).wait()

            def shuf(t, c2):
                jv = t * 16 + i16
                for bb in range(_BANDS):
                    for ss in range(8):
                        d = bb * 8 + ss
                        vals = plsc.load_gather(
                            rows_v.at[u], [jv, jnp.full((16,), d, jnp.int32)])
                        dst_v[u, bb, ss, pl.ds(t * 16, 16)] = vals
                return c2
            lax.fori_loop(0, 8, shuf, 0)

            pltpu.async_copy(dst_v.at[u], out_v.at[:, ot0 + i], so[u])

            @pl.when(g <= (n_blocks // 2 - 2))
            def _():
                gather(i + 2, u)
        return carry

    lax.fori_loop(0, n_blocks // 2, body, 0)
    for u in (0, 1):
        i = n_blocks - 2 + u
        pltpu.make_async_copy(dst_v.at[u], out_v.at[:, ot0 + i], so[u]).wait()


def kernel(feat_plane, mask_indices):
    tab_v = (feat_plane.reshape(_H, _WT, 128, _BANDS, 8)
             .transpose(0, 3, 1, 4, 2)
             .reshape(_H, _BANDS, _WT * 1024))
    idx = mask_indices.astype(jnp.int32)
    rm = _transpose_kernel(tab_v)
    out_v = _gather_kernel(rm, idx)
    return out_v.transpose(1, 3, 0, 2).reshape(_B, _D)


# speedup vs baseline: 2.6641x; 2.6641x over previous
import functools

import jax
import jax.numpy as jnp
from jax import lax
from jax.experimental import pallas as pl
from jax.experimental.pallas import tpu as pltpu
from jax.experimental.pallas import tpu_sc as plsc

_NC, _NS = 2, 16
_NW = _NC * _NS

_H, _W, _D = 1024, 1024, 48
_R = _H * _W
_B = 524288
_BANDS = _D // 8
_WT = _W // 128

_mesh = plsc.VectorSubcoreMesh(core_axis_name="c", subcore_axis_name="s")
_params = pltpu.CompilerParams(use_tc_tiling_on_sc=False,
                               needs_layout_passes=False)


def _wid():
    return lax.axis_index("s") * _NC + lax.axis_index("c")


@functools.partial(
    pl.kernel,
    out_type=jax.ShapeDtypeStruct((_R, _D), jnp.float32),
    mesh=_mesh,
    scratch_types=[
        pltpu.VMEM((2, _BANDS, 8, 128), jnp.float32),
        pltpu.VMEM((2, 128, _D), jnp.float32),
        pltpu.SemaphoreType.DMA,
        pltpu.SemaphoreType.DMA,
        pltpu.SemaphoreType.DMA,
        pltpu.SemaphoreType.DMA,
    ],
    compiler_params=_params,
)
def _transpose_kernel(tab_v, rm_out, src_v, dst_v, si0, si1, so0, so1):
    wid = _wid()
    h0 = wid * (_H // _NW)
    n_blocks = (_H // _NW) * _WT
    sin = (si0, si1)
    sout = (so0, so1)
    i16 = lax.iota(jnp.int32, 16)

    def src_slice(blk):
        h = h0 + blk // _WT
        wt = blk % _WT
        return tab_v.at[h, :, wt]

    for u in (0, 1):
        pltpu.async_copy(src_slice(u), src_v.at[u], sin[u])

    def body(g, carry):
        for u in (0, 1):
            i = 2 * g + u
            pltpu.make_async_copy(src_slice(i), src_v.at[u], sin[u]).wait()

            @pl.when(g > 0)
            def _():
                pltpu.make_async_copy(
                    dst_v.at[u],
                    rm_out.at[pl.ds((h0 * _WT + (i - 2)) * 128, 128)],
                    sout[u]).wait()

            def shuf(t, c2):
                rowv = t * 16 + i16
                for bb in range(_BANDS):
                    for ss in range(8):
                        vals = src_v[u, bb, ss, pl.ds(t * 16, 16)]
                        plsc.store_scatter(
                            dst_v.at[u],
                            [rowv, jnp.full((16,), bb * 8 + ss, jnp.int32)],
                            vals)
                return c2
            lax.fori_loop(0, 8, shuf, 0)

            pltpu.async_copy(
                dst_v.at[u],
                rm_out.at[pl.ds((h0 * _WT + i) * 128, 128)],
                sout[u])

            @pl.when(g <= (n_blocks // 2 - 2))
            def _():
                pltpu.async_copy(src_slice(i + 2), src_v.at[u], sin[u])
        return carry

    lax.fori_loop(0, n_blocks // 2, body, 0)
    for u in (0, 1):
        i = n_blocks - 2 + u
        pltpu.make_async_copy(
            dst_v.at[u],
            rm_out.at[pl.ds((h0 * _WT + i) * 128, 128)],
            sout[u]).wait()


@functools.partial(
    pl.kernel,
    out_type=jax.ShapeDtypeStruct((_BANDS, _B // 128, 8, 128), jnp.float32),
    mesh=_mesh,
    scratch_types=[
        pltpu.VMEM((_B // _NW,), jnp.int32),
        pltpu.VMEM((2, 128, _D), jnp.float32),
        pltpu.VMEM((2, _BANDS, 8, 128), jnp.float32),
        pltpu.SemaphoreType.DMA,
        pltpu.SemaphoreType.DMA,
        pltpu.SemaphoreType.DMA,
        pltpu.SemaphoreType.DMA,
    ],
    compiler_params=_params,
)
def _gather_kernel(rm_tab, idx_hbm, out_v, idx_v, rows_v, dst_v,
                   sg0, sg1, so0, so1):
    wid = _wid()
    b_per_w = _B // _NW
    n_blocks = b_per_w // 128
    ot0 = wid * n_blocks
    sg = (sg0, sg1)
    so = (so0, so1)
    pltpu.sync_copy(idx_hbm.at[pl.ds(wid * b_per_w, b_per_w)], idx_v)
    i16 = lax.iota(jnp.int32, 16)
    band_idx = []
    sub_idx = []
    for p in range(3):
        dv = p * 16 + i16
        band_idx.append(dv // 8)
        sub_idx.append(dv % 8)

    def gather(i, u):
        return pltpu.async_copy(
            rm_tab.at[idx_v.at[pl.ds(i * 128, 128)]], rows_v.at[u], sg[u])

    for u in (0, 1):
        gather(u, u)

    def body(g, carry):
        for u in (0, 1):
            i = 2 * g + u
            pltpu.make_async_copy(
                rm_tab.at[idx_v.at[pl.ds(i * 128, 128)]],
                rows_v.at[u], sg[u]).wait()

            @pl.when(g > 0)
            def _():
                pltpu.make_async_copy(
                    dst_v.at[u], out_v.at[:, ot0 + i - 2], so[u]).wait()

            def shuf(jc, c2):
                for jj in range(8):
                    j = jc * 8 + jj
                    jsplat = jnp.full((16,), 0, jnp.int32) + j
                    for p in range(3):
                        vals = rows_v[u, j, pl.ds(p * 16, 16)]
                        plsc.store_scatter(
                            dst_v.at[u],
                            [band_idx[p], sub_idx[p], jsplat],
                            vals)
                return c2
            lax.fori_loop(0, 16, shuf, 0)

            pltpu.async_copy(dst_v.at[u], out_v.at[:, ot0 + i], so[u])

            @pl.when(g <= (n_blocks // 2 - 2))
            def _():
                gather(i + 2, u)
        return carry

    lax.fori_loop(0, n_blocks // 2, body, 0)
    for u in (0, 1):
        i = n_blocks - 2 + u
        pltpu.make_async_copy(dst_v.at[u], out_v.at[:, ot0 + i], so[u]).wait()


def kernel(feat_plane, mask_indices):
    tab_v = feat_plane.reshape(_H, _WT, 128, _BANDS, 8).transpose(0, 3, 1, 4, 2)
    idx = mask_indices.astype(jnp.int32)
    rm = _transpose_kernel(tab_v)
    out_v = _gather_kernel(rm, idx)
    return out_v.transpose(1, 3, 0, 2).reshape(_B, _D)
